# baseline (speedup 1.0000x reference)
import functools
import jax
import jax.numpy as jnp
from jax import lax
from jax.experimental import pallas as pl
from jax.experimental.pallas import tpu as pltpu
from jax.experimental.pallas import tpu_sc as plsc

_BG = 32

_SC_CORES = 2
_SC_SUBCORES = 16
_SC_WORKERS = _SC_CORES * _SC_SUBCORES


def _sc_gather(table, sids):
    B = sids.shape[0]
    N, E = table.shape
    b_per_w = B // _SC_WORKERS
    mesh = plsc.VectorSubcoreMesh(core_axis_name="c", subcore_axis_name="s")

    @functools.partial(
        pl.kernel,
        mesh=mesh,
        out_type=jax.ShapeDtypeStruct((B, E), jnp.float32),
        scratch_types=[
            pltpu.VMEM((b_per_w,), jnp.int32),
            pltpu.VMEM((b_per_w, E), jnp.float32),
            pltpu.SemaphoreType.DMA,
        ],
    )
    def gather_k(table_hbm, idx_hbm, out_hbm, idx_v, rows_v, sem):
        wid = lax.axis_index("s") * _SC_CORES + lax.axis_index("c")
        base = wid * b_per_w
        pltpu.sync_copy(idx_hbm.at[pl.ds(base, b_per_w)], idx_v)
        pltpu.async_copy(table_hbm.at[idx_v], rows_v, sem).wait()
        pltpu.sync_copy(rows_v, out_hbm.at[pl.ds(base, b_per_w)])

    return gather_k(table, sids)


def _concat_body(emg_ref, emb_ref, out_ref):
    T = emg_ref.shape[1]
    F = emg_ref.shape[2]
    E = out_ref.shape[2] - F
    out_ref[:, :, :F] = emg_ref[...]
    rows = emb_ref[:, :E]
    out_ref[:, :, F:] = jnp.broadcast_to(rows[:, None, :], (_BG, T, E))


def _tc_concat(emg_features, embed):
    B, T, F = emg_features.shape
    E = 144 - F
    return pl.pallas_call(
        _concat_body,
        grid=(B // _BG,),
        in_specs=[
            pl.BlockSpec((_BG, T, F), lambda i: (i, 0, 0)),
            pl.BlockSpec((_BG, embed.shape[-1]), lambda i: (i, 0)),
        ],
        out_specs=pl.BlockSpec((_BG, T, F + E), lambda i: (i, 0, 0)),
        out_shape=jax.ShapeDtypeStruct((B, T, F + E), jnp.float32),
    )(emg_features, embed)


def kernel(emg_features, session_ids, table):
    sids = session_ids.astype(jnp.int32)
    table_p = jnp.pad(table, ((0, 0), (0, 128 - table.shape[1])))
    embed = _sc_gather(table_p, sids)
    return _tc_concat(emg_features, embed)
